# baseline (speedup 1.0000x reference)
import functools

import jax
import jax.numpy as jnp
from jax import lax
from jax.experimental import pallas as pl
from jax.experimental.pallas import tpu as pltpu
from jax.experimental.pallas import tpu_sc as plsc

_N_USERS = 25000
_N_NODES = 50000
_D = 64
_E = 800000

_NC = 2
_NS = 16
_NW = _NC * _NS
_CHUNK = 128
_E_PAD = 802816
_NCH = _E_PAD // _CHUNK
_W_CH = _NCH // _NW
_SEG_CAP = 200
_BLK = 28
_SBIAS = 16
_SECT = _SBIAS + _BLK * _CHUNK + 160
_HALF = _N_NODES // _NC
_ACC_ROWS = _HALF + 88
_ZR = 32
_CP_ROWS = 1560


def _route16(eb, vb, q, g, stc, strr, stv, f0, f1):
    sl16 = pl.ds(g * 16, 16)
    c16 = eb[q, 0, sl16]
    r16 = eb[q, 1, sl16]
    v16 = vb[pl.ds(q * _CHUNK + g * 16, 16)]
    lanes = lax.iota(jnp.int32, 16)
    for k in range(16):
        rot = (lanes + k) % 16
        crot = c16[rot]
        rrot = r16[rot]
        vrot = v16[rot]
        in0 = r16[k] < _HALF
        off = _SBIAS + jnp.where(in0, f0, _SECT + f1)
        stc[pl.ds(off, 16)] = crot
        strr[pl.ds(off, 16)] = rrot
        stv[pl.ds(off, 16)] = vrot
        one = jnp.where(in0, 1, 0).astype(jnp.int32)
        f0 = f0 + one
        f1 = f1 + (1 - one)
    return f0, f1


def _part_body(packed, valsh, pp, vp, cnts,
               ie0, ie1, iv0, iv1,
               stc, strr, stv, cbuf,
               si0, si1, sf):
    core = lax.axis_index("c")
    sid = lax.axis_index("s")
    w = core * _NS + sid
    q0 = w * _W_CH
    ie, iv, si = (ie0, ie1), (iv0, iv1), (si0, si1)

    def iload(blk, p):
        qb = q0 + blk * _BLK
        pltpu.async_copy(packed.at[pl.ds(qb, _BLK)], ie[p], si[p])
        pltpu.async_copy(valsh.at[pl.ds(qb * _CHUNK, _BLK * _CHUNK)], iv[p], si[p])

    def iwait(p):
        pltpu.make_async_copy(packed.at[pl.ds(0, _BLK)], ie[p], si[p]).wait()
        pltpu.make_async_copy(valsh.at[pl.ds(0, _BLK * _CHUNK)], iv[p], si[p]).wait()

    def flush(h, k, nchunks, move=True):
        seg = w * _SEG_CAP
        hb = h * _SECT + _SBIAS

        def body(j, carry):
            dst = seg + k + j
            sl = pl.ds(hb + j * _CHUNK, _CHUNK)
            pltpu.async_copy(stc.at[sl], pp.at[h, dst, 0], sf)
            pltpu.async_copy(strr.at[sl], pp.at[h, dst, 1], sf)
            pltpu.async_copy(stv.at[sl], vp.at[h, pl.ds(dst * _CHUNK, _CHUNK)], sf)
            return carry

        lax.fori_loop(0, nchunks, body, 0)

        def wbody(j, carry):
            sl0 = pl.ds(hb, _CHUNK)
            pltpu.make_async_copy(stc.at[sl0], pp.at[0, 0, 0], sf).wait()
            pltpu.make_async_copy(strr.at[sl0], pp.at[0, 0, 1], sf).wait()
            pltpu.make_async_copy(stv.at[sl0], vp.at[0, pl.ds(0, _CHUNK)], sf).wait()
            return carry

        lax.fori_loop(0, nchunks, wbody, 0)
        if move:
            base = nchunks * _CHUNK
            for i in range(_CHUNK // 16):
                sl_src = pl.ds(hb + base + i * 16, 16)
                sl_dst = pl.ds(hb + i * 16, 16)
                for ref in (stc, strr, stv):
                    ref[sl_dst] = ref[sl_src]

    def pad_nulls(h, rem):
        hb = h * _SECT + _SBIAS
        zi = jnp.zeros((16,), jnp.int32)
        zr = jnp.full((16,), h * _HALF, jnp.int32)
        zf = jnp.zeros((16,), jnp.float32)
        for i in range(_CHUNK // 16):
            sl = pl.ds(hb + rem + i * 16, 16)
            stc[sl] = zi
            strr[sl] = zr
            stv[sl] = zf

    iload(0, 0)
    f0, f1 = jnp.int32(0), jnp.int32(0)
    k0, k1 = jnp.int32(0), jnp.int32(0)
    for blk in range(_W_CH // _BLK):
        p = blk % 2
        if blk + 1 < _W_CH // _BLK:
            iload(blk + 1, 1 - p)
        iwait(p)

        def chunk_body(q, carry):
            def group_body(g, gc):
                return _route16(ie[p], iv[p], q, g, stc, strr, stv, gc[0], gc[1])
            return lax.fori_loop(0, _CHUNK // 16, group_body, carry)

        f0, f1 = lax.fori_loop(0, _BLK, chunk_body, (f0, f1))
        kb0 = f0 // _CHUNK
        kb1 = f1 // _CHUNK
        flush(0, k0, kb0)
        flush(1, k1, kb1)
        f0, k0 = f0 - kb0 * _CHUNK, k0 + kb0
        f1, k1 = f1 - kb1 * _CHUNK, k1 + kb1

    for h, f, k in ((0, f0, k0), (1, f1, k1)):
        pad_nulls(h, f)
        have_rem = jnp.where(f > 0, 1, 0)

        @pl.when(f > 0)
        def _():
            flush(h, k, 1, move=False)

        k = k + have_rem
        pad_nulls(h, jnp.int32(0))
        npad = (4 - k % 4) % 4
        npad = jnp.where(k + npad == 0, 4, npad)

        def nbody(j, carry):
            flush(h, k + j, 1, move=False)
            return carry

        lax.fori_loop(0, npad, nbody, 0)
        k = k + npad
        cbuf[pl.ds(0, 16)] = jnp.full((16,), 0, jnp.int32) + k
        pltpu.sync_copy(cbuf, cnts.at[h, w])


_part = functools.partial(
    pl.kernel,
    mesh=plsc.VectorSubcoreMesh(core_axis_name="c", subcore_axis_name="s"),
    compiler_params=pltpu.CompilerParams(use_tc_tiling_on_sc=False),
    out_type=(
        jax.ShapeDtypeStruct((2, _NW * _SEG_CAP, 2, _CHUNK), jnp.int32),
        jax.ShapeDtypeStruct((2, _NW * _SEG_CAP * _CHUNK), jnp.float32),
        jax.ShapeDtypeStruct((2, _NW, 16), jnp.int32),
    ),
    scratch_types=(
        [pltpu.VMEM((_BLK, 2, _CHUNK), jnp.int32) for _ in range(2)]
        + [pltpu.VMEM((_BLK * _CHUNK,), jnp.float32) for _ in range(2)]
        + [pltpu.VMEM((2 * _SECT,), jnp.int32),
           pltpu.VMEM((2 * _SECT,), jnp.int32),
           pltpu.VMEM((2 * _SECT,), jnp.float32),
           pltpu.VMEM((16,), jnp.int32)]
        + [pltpu.SemaphoreType.DMA for _ in range(3)]
    ),
)(_part_body)


def _scale_chunk(ebuf, vbuf, gbuf, rloc, base_row):
    for g in range(_CHUNK // 16):
        sl16 = pl.ds(g * 16, 16)
        r = ebuf[1, sl16]
        loc = r - base_row
        oob = (loc < 0) | (loc >= _HALF)
        rloc[sl16] = jnp.where(oob, _HALF, loc)
        vv = vbuf[sl16]
        for k in range(16):
            i = g * 16 + k
            v = vv[k]
            for j in range(_D // 16):
                sl = pl.ds(j * 16, 16)
                gbuf[i, sl] = gbuf[i, sl] * v


def _prop_body(table, pp, vp, cnts, out,
               ebuf0, ebuf1, vbuf0, vbuf1, gbuf0, gbuf1, rloc0, rloc1,
               zbuf, acc, cbuf, se0, se1, sg0, sg1, ss0, ss1):
    core = lax.axis_index("c")
    sid = lax.axis_index("s")
    base_row = core * _HALF
    ebuf, vbuf = (ebuf0, ebuf1), (vbuf0, vbuf1)
    gbuf, rloc = (gbuf0, gbuf1), (rloc0, rloc1)
    se, sg, ss = (se0, se1), (sg0, sg1), (ss0, ss1)

    def eload(qc, b):
        pltpu.async_copy(pp.at[core, qc], ebuf[b], se[b])
        pltpu.async_copy(vp.at[core, pl.ds(qc * _CHUNK, _CHUNK)], vbuf[b], se[b])

    def ewait(b):
        pltpu.make_async_copy(pp.at[0, 0], ebuf[b], se[b]).wait()
        pltpu.make_async_copy(vp.at[0, pl.ds(0, _CHUNK)], vbuf[b], se[b]).wait()

    def gather(b):
        pltpu.async_copy(table.at[ebuf[b].at[0]], gbuf[b], sg[b])

    def gwait(b):
        pltpu.make_async_copy(table.at[ebuf[b].at[0]], gbuf[b], sg[b]).wait()

    def swait(b):
        pltpu.make_async_copy(gbuf[b], acc.at[rloc[b]], ss[b]).wait()

    zero = jnp.zeros((16,), jnp.float32)
    for r in range(_ZR):
        for j in range(_D // 16):
            zbuf[r, pl.ds(j * 16, 16)] = zero

    def zloop(i, carry):
        pltpu.sync_copy(zbuf, acc.at[pl.ds(sid * 1568 + i * _ZR, _ZR)])
        return carry

    lax.fori_loop(0, 1568 // _ZR, zloop, 0)
    plsc.subcore_barrier()

    def run_segment(s):
        pltpu.sync_copy(cnts.at[core, s], cbuf)
        k_seg = cbuf[pl.ds(0, 16)][0]
        seg = s * _SEG_CAP

        def q_of(c):
            return seg + jnp.minimum(c, k_seg - 1)

        eload(q_of(0), 0)
        ewait(0)
        gather(0)
        eload(q_of(1), 1)

        def chunk_pair(i, carry):
            for b in (0, 1):
                c = 2 * i + b
                nb = 1 - b
                ewait(nb)

                @pl.when(c >= 1)
                def _():
                    swait(nb)

                gather(nb)
                gwait(b)
                _scale_chunk(ebuf[b], vbuf[b], gbuf[b], rloc[b], base_row)
                pltpu.async_copy(gbuf[b], acc.at[rloc[b]], ss[b], add=True)
                eload(q_of(c + 2), b)
            return carry

        lax.fori_loop(0, k_seg // 2, chunk_pair, 0)

        swait(1)
        gwait(0)
        ewait(1)

    run_segment(sid)
    run_segment(sid + _NS)
    plsc.subcore_barrier()

    pltpu.sync_copy(acc.at[pl.ds(sid * _CP_ROWS, _CP_ROWS)],
                    out.at[pl.ds(base_row + sid * _CP_ROWS, _CP_ROWS)])

    @pl.when(sid < 5)
    def _():
        tail = _NS * _CP_ROWS + sid * 8
        pltpu.sync_copy(acc.at[pl.ds(tail, 8)],
                        out.at[pl.ds(base_row + tail, 8)])


_prop = functools.partial(
    pl.kernel,
    mesh=plsc.VectorSubcoreMesh(core_axis_name="c", subcore_axis_name="s"),
    compiler_params=pltpu.CompilerParams(use_tc_tiling_on_sc=False),
    out_type=jax.ShapeDtypeStruct((_N_NODES, _D), jnp.float32),
    scratch_types=(
        [pltpu.VMEM((2, _CHUNK), jnp.int32) for _ in range(2)]
        + [pltpu.VMEM((_CHUNK,), jnp.float32) for _ in range(2)]
        + [pltpu.VMEM((_CHUNK, _D), jnp.float32) for _ in range(2)]
        + [pltpu.VMEM((_CHUNK,), jnp.int32) for _ in range(2)]
        + [pltpu.VMEM((_ZR, _D), jnp.float32),
           pltpu.VMEM_SHARED((_ACC_ROWS, _D), jnp.float32),
           pltpu.VMEM((16,), jnp.int32)]
        + [pltpu.SemaphoreType.DMA for _ in range(6)]
    ),
)(_prop_body)


def _mean_body(a, b, c, d, o):
    o[...] = (a[...] + b[...] + c[...] + d[...]) * 0.25


def _mean(x0, x1, x2, x3):
    blk = (1000, _D)
    spec = pl.BlockSpec(blk, lambda i: (i, 0))
    return pl.pallas_call(
        _mean_body,
        grid=(_N_NODES // blk[0],),
        in_specs=[spec] * 4,
        out_specs=spec,
        out_shape=jax.ShapeDtypeStruct((_N_NODES, _D), jnp.float32),
    )(x0, x1, x2, x3)


def kernel(user_emb, item_emb, edge_index, edge_values):
    rows = jnp.asarray(edge_index[0], jnp.int32)
    cols = jnp.asarray(edge_index[1], jnp.int32)
    vals = edge_values.astype(jnp.float32)
    pad = _E_PAD - _E
    rows = jnp.concatenate([rows, jnp.zeros((pad,), jnp.int32)])
    cols = jnp.concatenate([cols, jnp.zeros((pad,), jnp.int32)])
    vals = jnp.concatenate([vals, jnp.zeros((pad,), jnp.float32)])
    packed = jnp.stack([cols.reshape(_NCH, _CHUNK),
                        rows.reshape(_NCH, _CHUNK)], axis=1)

    pp, vp, cnts = _part(packed, vals)
    x0 = jnp.concatenate([user_emb, item_emb], axis=0)
    x1 = _prop(x0, pp, vp, cnts)
    x2 = _prop(x1, pp, vp, cnts)
    x3 = _prop(x2, pp, vp, cnts)
    m = _mean(x0, x1, x2, x3)
    return m[:_N_USERS], m[_N_USERS:]
